# Initial kernel scaffold; baseline (speedup 1.0000x reference)
import jax
import jax.numpy as jnp
from jax.experimental import pallas as pl


def kernel(struct_h, cont_h, Wq, Wk, Wv, W1, b1, W2, b2, ln_w, ln_b):
    raise NotImplementedError("write your pallas kernel here")


# speedup vs baseline: 15.7319x; 15.7319x over previous
import functools

import jax
import jax.numpy as jnp
import numpy as np
from jax.experimental import pallas as pl

_N, _D, _H = 100000, 128, 64
_INV_TEMP = 1.0 / float(np.sqrt(_D))
_BLOCK = 1000


def _ffn_ln(h, w1t, b1, w2t, b2, lnw, lnb):
    y = jnp.maximum(jnp.dot(h, w1t, preferred_element_type=jnp.float32) + b1, 0.0)
    y = jnp.dot(y, w2t, preferred_element_type=jnp.float32) + b2
    r = y + h
    mu = jnp.mean(r, axis=-1, keepdims=True)
    c = r - mu
    var = jnp.mean(c * c, axis=-1, keepdims=True)
    return c * jax.lax.rsqrt(var + 1e-6) * lnw + lnb


def _body(xs_ref, xc_ref, wqkv_ref, w1t_ref, b1_ref, w2t_ref, b2_ref,
          lnw_ref, lnb_ref, os_ref, oc_ref):
    xs = xs_ref[...]
    xc = xc_ref[...]
    wqkv = wqkv_ref[...]

    qkv_s = jnp.dot(xs, wqkv, preferred_element_type=jnp.float32)
    qkv_c = jnp.dot(xc, wqkv, preferred_element_type=jnp.float32)
    qs, ks, vs = qkv_s[:, :_D], qkv_s[:, _D:2 * _D], qkv_s[:, 2 * _D:]
    qc, kc, vc = qkv_c[:, :_D], qkv_c[:, _D:2 * _D], qkv_c[:, 2 * _D:]

    s00 = jnp.sum(qs * ks, axis=-1, keepdims=True) * _INV_TEMP
    s01 = jnp.sum(qs * kc, axis=-1, keepdims=True) * _INV_TEMP
    s10 = jnp.sum(qc * ks, axis=-1, keepdims=True) * _INV_TEMP
    s11 = jnp.sum(qc * kc, axis=-1, keepdims=True) * _INV_TEMP

    m0 = jnp.maximum(s00, s01)
    e00 = jnp.exp(s00 - m0)
    e01 = jnp.exp(s01 - m0)
    d0 = e00 + e01
    m1 = jnp.maximum(s10, s11)
    e10 = jnp.exp(s10 - m1)
    e11 = jnp.exp(s11 - m1)
    d1 = e10 + e11

    hs = (e00 * vs + e01 * vc) / d0
    hc = (e10 * vs + e11 * vc) / d1

    w1t = w1t_ref[...]
    b1 = b1_ref[...]
    w2t = w2t_ref[...]
    b2 = b2_ref[...]
    lnw = lnw_ref[...]
    lnb = lnb_ref[...]
    os_ref[...] = _ffn_ln(hs, w1t, b1, w2t, b2, lnw, lnb)
    oc_ref[...] = _ffn_ln(hc, w1t, b1, w2t, b2, lnw, lnb)


@functools.partial(jax.jit, static_argnames=("interpret",))
def kernel(struct_h, cont_h, Wq, Wk, Wv, W1, b1, W2, b2, ln_w, ln_b,
           interpret=False):
    wqkv = jnp.concatenate([Wq.T, Wk.T, Wv.T], axis=1)
    w1t = W1.T
    w2t = W2.T
    b1r = b1.reshape(1, _H)
    b2r = b2.reshape(1, _D)
    lnw = ln_w.reshape(1, _D)
    lnb = ln_b.reshape(1, _D)

    grid = (_N // _BLOCK,)
    row_spec = pl.BlockSpec((_BLOCK, _D), lambda i: (i, 0))
    full = lambda shape: pl.BlockSpec(shape, lambda i: (0,) * len(shape))

    struct_o, cont_o = pl.pallas_call(
        _body,
        grid=grid,
        in_specs=[
            row_spec,
            row_spec,
            full((_D, 3 * _D)),
            full((_D, _H)),
            full((1, _H)),
            full((_H, _D)),
            full((1, _D)),
            full((1, _D)),
            full((1, _D)),
        ],
        out_specs=[row_spec, row_spec],
        out_shape=[
            jax.ShapeDtypeStruct((_N, _D), jnp.float32),
            jax.ShapeDtypeStruct((_N, _D), jnp.float32),
        ],
        interpret=interpret,
    )(struct_h, cont_h, wqkv, w1t, b1r, w2t, b2r, lnw, lnb)
    return (struct_o, cont_o)
